# initial kernel scaffold (unmeasured)
import functools

import jax
import jax.numpy as jnp
from jax import lax
from jax.experimental import pallas as pl
from jax.experimental.pallas import tpu as pltpu

N_DEV = 32
N_TOK = 1024
D_IN = 256
D_OUT = 512
E_PER = 4
CAP = 6
SLOT = 8
M_ROWS = N_TOK // N_DEV
COMPACT = E_PER * SLOT


def kernel(x, router_W, route_idx, expert_W):
    del router_W

    def body(x_ref, idx_ref, w_ref, out_ref, partial_ref, comm_ref,
             send_sems, recv_sems):
        my = lax.axis_index("i")

        idx = idx_ref[:, 0]

        ii = lax.broadcasted_iota(jnp.int32, (N_TOK, N_TOK), 0)
        jj = lax.broadcasted_iota(jnp.int32, (N_TOK, N_TOK), 1)
        same = (idx[:, None] == idx[None, :]) & (jj <= ii)
        rank = jnp.sum(same.astype(jnp.float32), axis=1)
        keep = rank <= CAP

        le = idx - E_PER * my
        mine = (le >= 0) & (le < E_PER)

        cols = lax.broadcasted_iota(jnp.int32, (N_TOK, COMPACT), 1)
        col_le = cols // SLOT
        col_s = cols % SLOT
        sel = (mine & keep)[:, None] & (col_le == le[:, None]) & \
              (col_s == (rank.astype(jnp.int32) - 1)[:, None])
        M = sel.astype(jnp.float32)

        compact_x = lax.dot_general(
            M, x_ref[:, :], (((0,), (0,)), ((), ())),
            preferred_element_type=jnp.float32)

        ys = []
        for e in range(E_PER):
            ys.append(jnp.dot(compact_x[e * SLOT:(e + 1) * SLOT, :],
                              w_ref[e, :, :],
                              preferred_element_type=jnp.float32))
        y_compact = jnp.concatenate(ys, axis=0)

        partial_ref[:, :] = jnp.dot(M, y_compact,
                                    preferred_element_type=jnp.float32)

        comm_ref[0, :, :] = partial_ref[pl.ds(my * M_ROWS, M_ROWS), :]

        rdmas = []
        for k in range(1, N_DEV):
            dst = lax.rem(my + k, N_DEV)
            rdma = pltpu.make_async_remote_copy(
                src_ref=partial_ref.at[pl.ds(dst * M_ROWS, M_ROWS), :],
                dst_ref=comm_ref.at[k],
                send_sem=send_sems.at[k],
                recv_sem=recv_sems.at[k],
                device_id=(dst,),
                device_id_type=pl.DeviceIdType.MESH,
            )
            rdma.start()
            rdmas.append(rdma)

        for rdma in rdmas:
            rdma.wait_recv()
        out_ref[:, :] = jnp.sum(comm_ref[:, :, :], axis=0)

        for rdma in rdmas:
            rdma.wait_send()

    return pl.pallas_call(
        body,
        out_shape=jax.ShapeDtypeStruct((M_ROWS, D_OUT), jnp.float32),
        in_specs=[
            pl.BlockSpec(memory_space=pltpu.VMEM),
            pl.BlockSpec(memory_space=pltpu.VMEM),
            pl.BlockSpec(memory_space=pltpu.VMEM),
        ],
        out_specs=pl.BlockSpec(memory_space=pltpu.VMEM),
        scratch_shapes=[
            pltpu.VMEM((N_TOK, D_OUT), jnp.float32),
            pltpu.VMEM((N_DEV, M_ROWS, D_OUT), jnp.float32),
            pltpu.SemaphoreType.DMA((N_DEV,)),
            pltpu.SemaphoreType.DMA((N_DEV,)),
        ],
        compiler_params=pltpu.CompilerParams(collective_id=0),
    )(x, route_idx, expert_W)


# baseline (device time: 42517 ns/iter reference)
import functools

import jax
import jax.numpy as jnp
from jax import lax
from jax.experimental import pallas as pl
from jax.experimental.pallas import tpu as pltpu

N_DEV = 32
N_TOK = 1024
D_IN = 256
D_OUT = 512
E_PER = 4
CAP = 6
SLOT = 8
M_ROWS = N_TOK // N_DEV
COMPACT = E_PER * SLOT


def kernel(x, router_W, route_idx, expert_W):
    del router_W

    def body(x_ref, idx_ref, w_ref, out_ref, partial_ref, comm_ref,
             send_sems, recv_sems):
        my = lax.axis_index("i")

        idx = idx_ref[:, 0]

        ii = lax.broadcasted_iota(jnp.int32, (N_TOK, N_TOK), 0)
        jj = lax.broadcasted_iota(jnp.int32, (N_TOK, N_TOK), 1)
        same = (idx[:, None] == idx[None, :]) & (jj <= ii)
        rank = jnp.sum(same.astype(jnp.float32), axis=1)
        keep = rank <= CAP

        le = idx - E_PER * my
        mine = (le >= 0) & (le < E_PER)

        cols = lax.broadcasted_iota(jnp.int32, (N_TOK, COMPACT), 1)
        col_le = cols // SLOT
        col_s = cols % SLOT
        sel = (mine & keep)[:, None] & (col_le == le[:, None]) & \
              (col_s == (rank.astype(jnp.int32) - 1)[:, None])
        M = sel.astype(jnp.float32)

        compact_x = lax.dot_general(
            M, x_ref[:, :], (((0,), (0,)), ((), ())),
            preferred_element_type=jnp.float32)

        ys = []
        for e in range(E_PER):
            ys.append(jnp.dot(compact_x[e * SLOT:(e + 1) * SLOT, :],
                              w_ref[e, :, :],
                              preferred_element_type=jnp.float32))
        y_compact = jnp.concatenate(ys, axis=0)

        partial_ref[:, :] = jnp.dot(M, y_compact,
                                    preferred_element_type=jnp.float32)

        comm_ref[0, :, :] = partial_ref[pl.ds(my * M_ROWS, M_ROWS), :]

        rdmas = []
        for k in range(1, N_DEV):
            dst = lax.rem(my + k, N_DEV)
            rdma = pltpu.make_async_remote_copy(
                src_ref=partial_ref.at[pl.ds(dst * M_ROWS, M_ROWS), :],
                dst_ref=comm_ref.at[k],
                send_sem=send_sems.at[k],
                recv_sem=recv_sems.at[k],
                device_id=(dst,),
                device_id_type=pl.DeviceIdType.MESH,
            )
            rdma.start()
            rdmas.append(rdma)

        for rdma in rdmas:
            rdma.wait_recv()
        out_ref[:, :] = jnp.sum(comm_ref[:, :, :], axis=0)

        for rdma in rdmas:
            rdma.wait_send()

    return pl.pallas_call(
        body,
        out_shape=jax.ShapeDtypeStruct((M_ROWS, D_OUT), jnp.float32),
        in_specs=[
            pl.BlockSpec(memory_space=pltpu.VMEM),
            pl.BlockSpec(memory_space=pltpu.VMEM),
            pl.BlockSpec(memory_space=pltpu.VMEM),
        ],
        out_specs=pl.BlockSpec(memory_space=pltpu.VMEM),
        scratch_shapes=[
            pltpu.VMEM((N_TOK, D_OUT), jnp.float32),
            pltpu.VMEM((N_DEV, M_ROWS, D_OUT), jnp.float32),
            pltpu.SemaphoreType.DMA((N_DEV,)),
            pltpu.SemaphoreType.DMA((N_DEV,)),
        ],
    )(x, route_idx, expert_W)
